# baseline (speedup 1.0000x reference)
import jax
import jax.numpy as jnp
from jax.experimental import pallas as pl
from jax.experimental.pallas import tpu as pltpu

B, S = 32, 128
V, E, H = 256, 64, 512
CS = 32
NC = S // CS
G4 = 4 * H


def _dotT(a, w):
    return jax.lax.dot_general(a, w, (((1,), (1,)), ((), ())),
                               preferred_element_type=jnp.float32)


def _gates(z, c_prev):
    i = jax.nn.sigmoid(z[:, 0:H])
    f = jax.nn.sigmoid(z[:, H:2 * H])
    g = jnp.tanh(z[:, 2 * H:3 * H])
    o = jax.nn.sigmoid(z[:, 3 * H:4 * H])
    cn = f * c_prev + i * g
    hn = o * jnp.tanh(cn)
    return hn, cn


def _lstm_fwd_kernel(xT_ref, emb_ref, Wih0_ref, Whh0_ref, b0_ref,
                     Wih1_ref, Whh1_ref, b1_ref, Wout_ref, bout_ref,
                     logp_ref, h_out_ref, c_out_ref,
                     P_ref, y1_ref):
    f32 = jnp.float32
    emb = emb_ref[...]
    b0 = b0_ref[...]
    b1 = b1_ref[...]
    iota_v = jax.lax.broadcasted_iota(jnp.int32, (CS, B, V), 2)

    def chunk(c, carry):
        h0, c0, h1, c1 = carry
        xc = xT_ref[pl.ds(c * CS, CS), :]
        oh = (xc[:, :, None] == iota_v).astype(f32).reshape(CS * B, V)
        xe = jnp.dot(oh, emb, preferred_element_type=f32)
        P_ref[...] = _dotT(xe, Wih0_ref[...]) + b0

        def step(s, hc):
            h0, c0, h1, c1 = hc
            z0 = P_ref[pl.ds(s * B, B), :] + _dotT(h0, Whh0_ref[...])
            h0, c0 = _gates(z0, c0)
            z1 = _dotT(h0, Wih1_ref[...]) + _dotT(h1, Whh1_ref[...]) + b1
            h1, c1 = _gates(z1, c1)
            y1_ref[pl.ds((c * CS + s) * B, B), :] = h1
            return (h0, c0, h1, c1)

        return jax.lax.fori_loop(0, CS, step, (h0, c0, h1, c1), unroll=8)

    init = (jnp.zeros((B, H), f32),) * 4
    h0, c0, h1, c1 = jax.lax.fori_loop(0, NC, chunk, init)
    h_out_ref[0, :, :] = h0
    c_out_ref[0, :, :] = c0
    h_out_ref[1, :, :] = h1
    c_out_ref[1, :, :] = c1

    logits = _dotT(y1_ref[...], Wout_ref[...]) + bout_ref[...]
    m = jnp.max(logits, axis=-1, keepdims=True)
    lse = jnp.log(jnp.sum(jnp.exp(logits - m), axis=-1, keepdims=True)) + m
    logp_ref[...] = logits - lse


def kernel(x, emb, Wih0, Whh0, bih0, bhh0, Wih1, Whh1, bih1, bhh1, W_out, b_out):
    xT = x.T
    b0 = (bih0 + bhh0).reshape(1, G4)
    b1 = (bih1 + bhh1).reshape(1, G4)
    bout = b_out.reshape(1, V)

    logp_t, h_out, c_out = pl.pallas_call(
        _lstm_fwd_kernel,
        out_shape=[
            jax.ShapeDtypeStruct((S * B, V), jnp.float32),
            jax.ShapeDtypeStruct((2, B, H), jnp.float32),
            jax.ShapeDtypeStruct((2, B, H), jnp.float32),
        ],
        scratch_shapes=[
            pltpu.VMEM((CS * B, G4), jnp.float32),
            pltpu.VMEM((S * B, H), jnp.float32),
        ],
    )(xT, emb, Wih0, Whh0, b0, Wih1, Whh1, b1, W_out, bout)

    next_logp = logp_t.reshape(S, B, V).transpose(1, 0, 2).reshape(B * S, V)
    return (next_logp, (h_out, c_out))


# speedup vs baseline: 1.3281x; 1.3281x over previous
import jax
import jax.numpy as jnp
from jax.experimental import pallas as pl
from jax.experimental.pallas import tpu as pltpu

B, S = 32, 128
V, E, H = 256, 64, 512
CS = 16
NC = S // CS
G4 = 4 * H
UNROLL = 8


def _dotT(a, w):
    return jax.lax.dot_general(a, w, (((1,), (1,)), ((), ())),
                               preferred_element_type=jnp.float32)


def _gates(z, c_prev):
    i = jax.nn.sigmoid(z[:, 0:H])
    f = jax.nn.sigmoid(z[:, H:2 * H])
    g = jnp.tanh(z[:, 2 * H:3 * H])
    o = jax.nn.sigmoid(z[:, 3 * H:4 * H])
    cn = f * c_prev + i * g
    hn = o * jnp.tanh(cn)
    return hn, cn


def _lstm_fwd_kernel(xT_ref, emb_ref, Wih0_ref, Whh0_ref, b0_ref,
                     Wih1_ref, Whh1_ref, b1_ref, Wout_ref, bout_ref,
                     logp_ref, h_out_ref, c_out_ref,
                     P0_ref, P1_ref, y0_ref, y1_ref):
    f32 = jnp.float32
    emb = emb_ref[...]
    b0 = b0_ref[...]
    b1 = b1_ref[...]
    iota_v = jax.lax.broadcasted_iota(jnp.int32, (CS, B, V), 2)

    def project0(c):
        xc = xT_ref[pl.ds(c * CS, CS), :]
        oh = (xc[:, :, None] == iota_v).astype(f32).reshape(CS * B, V)
        xe = jnp.dot(oh, emb, preferred_element_type=f32)
        P0_ref[...] = _dotT(xe, Wih0_ref[...]) + b0

    def project1(c):
        yc = y0_ref[pl.ds(c * CS * B, CS * B), :]
        P1_ref[...] = _dotT(yc, Wih1_ref[...]) + b1

    def step0(c, s, h0, c0):
        z0 = P0_ref[pl.ds(s * B, B), :] + _dotT(h0, Whh0_ref[...])
        h0, c0 = _gates(z0, c0)
        y0_ref[pl.ds((c * CS + s) * B, B), :] = h0
        return h0, c0

    def step1(c, s, h1, c1):
        z1 = P1_ref[pl.ds(s * B, B), :] + _dotT(h1, Whh1_ref[...])
        h1, c1 = _gates(z1, c1)
        y1_ref[pl.ds((c * CS + s) * B, B), :] = h1
        return h1, c1

    z = jnp.zeros((B, H), f32)

    project0(0)

    def pro_step(s, hc):
        h0, c0 = hc
        return step0(0, s, h0, c0)

    h0, c0 = jax.lax.fori_loop(0, CS, pro_step, (z, z), unroll=UNROLL)

    def fused_chunk(c, carry):
        h0, c0, h1, c1 = carry
        project0(c)
        project1(c - 1)

        def step(s, hc):
            h0, c0, h1, c1 = hc
            h0, c0 = step0(c, s, h0, c0)
            h1, c1 = step1(c - 1, s, h1, c1)
            return (h0, c0, h1, c1)

        return jax.lax.fori_loop(0, CS, step, (h0, c0, h1, c1), unroll=UNROLL)

    h0, c0, h1, c1 = jax.lax.fori_loop(1, NC, fused_chunk, (h0, c0, z, z))
    h_out_ref[0, :, :] = h0
    c_out_ref[0, :, :] = c0

    project1(NC - 1)

    def epi_step(s, hc):
        h1, c1 = hc
        return step1(NC - 1, s, h1, c1)

    h1, c1 = jax.lax.fori_loop(0, CS, epi_step, (h1, c1), unroll=UNROLL)
    h_out_ref[1, :, :] = h1
    c_out_ref[1, :, :] = c1

    logits = _dotT(y1_ref[...], Wout_ref[...]) + bout_ref[...]
    m = jnp.max(logits, axis=-1, keepdims=True)
    lse = jnp.log(jnp.sum(jnp.exp(logits - m), axis=-1, keepdims=True)) + m
    logp_ref[...] = logits - lse


def kernel(x, emb, Wih0, Whh0, bih0, bhh0, Wih1, Whh1, bih1, bhh1, W_out, b_out):
    xT = x.T
    b0 = (bih0 + bhh0).reshape(1, G4)
    b1 = (bih1 + bhh1).reshape(1, G4)
    bout = b_out.reshape(1, V)

    logp_t, h_out, c_out = pl.pallas_call(
        _lstm_fwd_kernel,
        out_shape=[
            jax.ShapeDtypeStruct((S * B, V), jnp.float32),
            jax.ShapeDtypeStruct((2, B, H), jnp.float32),
            jax.ShapeDtypeStruct((2, B, H), jnp.float32),
        ],
        scratch_shapes=[
            pltpu.VMEM((CS * B, G4), jnp.float32),
            pltpu.VMEM((CS * B, G4), jnp.float32),
            pltpu.VMEM((S * B, H), jnp.float32),
            pltpu.VMEM((S * B, H), jnp.float32),
        ],
    )(xT, emb, Wih0, Whh0, b0, Wih1, Whh1, b1, W_out, bout)

    next_logp = logp_t.reshape(S, B, V).transpose(1, 0, 2).reshape(B * S, V)
    return (next_logp, (h_out, c_out))
